# baseline (speedup 1.0000x reference)
import jax
import jax.numpy as jnp
from jax.experimental import pallas as pl

_S = 64.0
_M2 = 0.4
_VB = 896


def _body(labs_ref, x_ref, o_ref):
    v0 = pl.program_id(0) * _VB
    x = x_ref[...]
    labs = labs_ref[...]
    labs = jnp.where(labs >= 0, labs, -(2**30))
    vio = jax.lax.broadcasted_iota(jnp.int32, x.shape, 0) + v0
    mask = vio == labs
    t = jnp.sum(jnp.where(mask, x, 0.0), axis=0, keepdims=True)
    y = (t / (1.0 + jnp.sqrt(jnp.maximum(1.0 - t * t, 0.0))) - _M2) * _S
    o_ref[...] = jnp.where(mask, y, x * _S)


def kernel(logits, labels):
    B, V = logits.shape
    lT = jnp.swapaxes(logits, 0, 1)
    labs2 = labels.reshape(1, B)
    outT = pl.pallas_call(
        _body,
        grid=(pl.cdiv(V, _VB),),
        in_specs=[
            pl.BlockSpec((1, B), lambda i: (0, 0)),
            pl.BlockSpec((_VB, B), lambda i: (i, 0)),
        ],
        out_specs=pl.BlockSpec((_VB, B), lambda i: (i, 0)),
        out_shape=jax.ShapeDtypeStruct((V, B), jnp.float32),
    )(labs2, lT)
    return jnp.swapaxes(outT, 0, 1)
